# Initial kernel scaffold; baseline (speedup 1.0000x reference)
import jax
import jax.numpy as jnp
from jax.experimental import pallas as pl


def kernel(stimulus_set, kernel_gate_weights, table, w0, w1):
    raise NotImplementedError("write your pallas kernel here")


# speedup vs baseline: 11.6267x; 11.6267x over previous
import functools

import jax
import jax.numpy as jnp
from jax import lax
from jax.experimental import pallas as pl
from jax.experimental.pallas import tpu as pltpu
from jax.experimental.pallas import tpu_sc as plsc

B = 16384
N_STIMULI = 30
N_DIM = 10
N_REF = 4
NV = N_STIMULI + 1

NC = 2
NS = 16
L = 16
NW = NC * NS
TPW = B // NW
CHUNKS = TPW // L


def _sim_body(table_ref, w0_ref, w1_ref, out_ref):
    t = table_ref[...]
    z1 = t[:, None, :]
    z2 = t[None, :, :]
    sq = (z1 - z2) * (z1 - z2)
    for g in range(2):
        w = (w0_ref if g == 0 else w1_ref)[...]
        d2 = jnp.sum(sq * w[None, :, :], axis=-1)
        out_ref[g, :, :] = jnp.exp(-jnp.sqrt(d2 + 1e-12))


_sim_tables = pl.pallas_call(
    _sim_body,
    out_shape=jax.ShapeDtypeStruct((2, NV, NV), jnp.float32),
)


def _sc_body(sim_hbm, ss_hbm, gate_hbm, out_hbm, sim_v, ss_v, gate_v, out_v):
    cid = lax.axis_index("c")
    sid = lax.axis_index("s")
    wid = sid * NC + cid
    base = wid * TPW

    pltpu.sync_copy(sim_hbm, sim_v)
    pltpu.sync_copy(ss_hbm.at[pl.ds(base * 5, TPW * 5)], ss_v)
    pltpu.sync_copy(gate_hbm.at[pl.ds(base, TPW)], gate_v)

    lane = lax.iota(jnp.int32, L)

    def chunk(g, carry):
        rows = lane + g * L
        row5 = rows * 5
        gt = gate_v[pl.ds(g * L, L)]
        q = plsc.load_gather(ss_v, [row5])
        gq = gt * (NV * NV) + q * NV
        s_vals = []
        for j in range(N_REF):
            rj = plsc.load_gather(ss_v, [row5 + (1 + j)])
            s_vals.append(plsc.load_gather(sim_v, [gq + rj]))
        tot = (s_vals[0] + s_vals[1]) + (s_vals[2] + s_vals[3])
        inv = 1.0 / tot
        row4 = rows * 4
        for j in range(N_REF):
            plsc.store_scatter(out_v, [row4 + j], s_vals[j] * inv)
        return carry

    lax.fori_loop(0, CHUNKS, chunk, 0)
    pltpu.sync_copy(out_v, out_hbm.at[pl.ds(base * 4, TPW * 4)])


@functools.lru_cache(maxsize=1)
def _sc_rank():
    return pl.kernel(
        _sc_body,
        out_type=jax.ShapeDtypeStruct((B * N_REF,), jnp.float32),
        mesh=plsc.VectorSubcoreMesh(core_axis_name="c", subcore_axis_name="s",
                                    num_cores=NC, num_subcores=NS),
        compiler_params=pltpu.CompilerParams(needs_layout_passes=False),
        scratch_types=[
            pltpu.VMEM((2 * NV * NV,), jnp.float32),
            pltpu.VMEM((TPW * (1 + N_REF),), jnp.int32),
            pltpu.VMEM((TPW,), jnp.int32),
            pltpu.VMEM((TPW * N_REF,), jnp.float32),
        ],
    )


def kernel(stimulus_set, kernel_gate_weights, table, w0, w1):
    sim = _sim_tables(table, w0.reshape(1, N_DIM), w1.reshape(1, N_DIM))
    ss = stimulus_set.astype(jnp.int32).reshape(B * (1 + N_REF))
    gate = kernel_gate_weights.astype(jnp.int32)
    out = _sc_rank()(sim.reshape(2 * NV * NV), ss, gate)
    return out.reshape(B, N_REF)
